# Initial kernel scaffold; baseline (speedup 1.0000x reference)
import jax
import jax.numpy as jnp
from jax.experimental import pallas as pl


def kernel(x, n_instances, W1, b1, W2, b2, W3, b3):
    raise NotImplementedError("write your pallas kernel here")


# speedup vs baseline: 7.3393x; 7.3393x over previous
import functools

import jax
import jax.numpy as jnp
from jax import lax
from jax.experimental import pallas as pl
from jax.experimental.pallas import tpu as pltpu
from jax.experimental.pallas import tpu_sc as plsc

_T = 32768
_D = 512
_H = 1024
_B = 16
_TR = 1024
_NT = _T // _TR
_SROWS = _T // 128
_CHUNK = 16
_SPAN = 2080
_NSTEPS = _SPAN // _CHUNK


def _tc_body(nv_ref, x_ref, w1_ref, b1_ref, w2_ref, w3_ref, b2_ref, b3_ref,
             s_ref, consts_ref, wrow_ref):
    i = pl.program_id(0)

    @pl.when(i == 0)
    def _prologue():
        wrow_ref[...] = lax.dot_general(
            w3_ref[...], w2_ref[...], (((0,), (1,)), ((), ())),
            preferred_element_type=jnp.float32)
        c = jnp.dot(b2_ref[...], w3_ref[...],
                    preferred_element_type=jnp.float32)
        lanes = lax.broadcasted_iota(jnp.int32, (1, 128), 1)
        consts_ref[...] = jnp.where(
            lanes == 0, c[0, 0],
            jnp.where(lanes == 1, b3_ref[0, 0], 0.0))

    @pl.when(i < nv_ref[0])
    def _tile():
        xb = x_ref[...].astype(jnp.bfloat16)
        g = jnp.dot(xb, w1_ref[...], preferred_element_type=jnp.float32)
        h = jnp.maximum(g + b1_ref[...], 0.0)
        s = jnp.sum(h * wrow_ref[...], axis=1)
        s_ref[...] = s.reshape(_TR // 128, 128)


def _tc_stage(nv_arr, x, W1b, b1r, W2, W3, b2r, b3p):
    grid_spec = pltpu.PrefetchScalarGridSpec(
        num_scalar_prefetch=1,
        grid=(_NT,),
        in_specs=[
            pl.BlockSpec((_TR, _D),
                         lambda i, nv: (jnp.maximum(jnp.minimum(i, nv[0] - 1), 0), 0)),
            pl.BlockSpec((_D, _H), lambda i, nv: (0, 0)),
            pl.BlockSpec((1, _H), lambda i, nv: (0, 0)),
            pl.BlockSpec((_H, _D), lambda i, nv: (0, 0)),
            pl.BlockSpec((_D, 1), lambda i, nv: (0, 0)),
            pl.BlockSpec((1, _D), lambda i, nv: (0, 0)),
            pl.BlockSpec((1, 128), lambda i, nv: (0, 0)),
        ],
        out_specs=[
            pl.BlockSpec((_TR // 128, 128),
                         lambda i, nv: (jnp.maximum(jnp.minimum(i, nv[0] - 1), 0), 0)),
            pl.BlockSpec((1, 128), lambda i, nv: (0, 0)),
        ],
        scratch_shapes=[pltpu.VMEM((1, _H), jnp.float32)],
    )
    return pl.pallas_call(
        _tc_body,
        grid_spec=grid_spec,
        out_shape=[
            jax.ShapeDtypeStruct((_SROWS, 128), jnp.float32),
            jax.ShapeDtypeStruct((1, 128), jnp.float32),
        ],
    )(nv_arr, x, W1b, b1r, W2, W3, b2r, b3p)


def _sc_body(s_hbm, n_hbm, c_hbm, out_hbm, buf, nbuf, cbuf, obuf):
    wid = lax.axis_index("s") * 2 + lax.axis_index("c")

    @pl.when(wid < _B)
    def _():
        pltpu.sync_copy(n_hbm, nbuf)
        pltpu.sync_copy(c_hbm.at[pl.ds(0, _CHUNK)], cbuf)
        nvec = nbuf[...]
        lane = lax.iota(jnp.int32, _CHUNK)
        start = jnp.sum(jnp.where(lane < wid, nvec, 0))
        nb = jnp.sum(jnp.where(lane == wid, nvec, 0))
        end = start + nb
        a0 = jnp.minimum((start // _CHUNK) * _CHUNK, _T - _SPAN)
        a0 = pl.multiple_of(a0, _CHUNK)
        pltpu.sync_copy(s_hbm.at[pl.ds(a0, _SPAN)], buf)
        cvec = cbuf[...]
        cval = jnp.sum(jnp.where(lane == 0, cvec, 0.0))
        b3val = jnp.sum(jnp.where(lane == 1, cvec, 0.0))

        def step(k, acc):
            v = buf[pl.ds(k * _CHUNK, _CHUNK)]
            pos = lane + a0 + k * _CHUNK
            m = (pos >= start) & (pos < end)
            return acc + jnp.where(m, v, 0.0)

        acc = lax.fori_loop(0, _NSTEPS, step, jnp.zeros((_CHUNK,), jnp.float32))
        val = jnp.sum(acc) + nb.astype(jnp.float32) * cval + b3val
        obuf[...] = jnp.zeros((_CHUNK,), jnp.float32) + val
        pltpu.sync_copy(obuf, out_hbm.at[wid])


@functools.cache
def _sc_stage():
    return pl.kernel(
        _sc_body,
        out_type=jax.ShapeDtypeStruct((_B, _CHUNK), jnp.float32),
        mesh=plsc.VectorSubcoreMesh(core_axis_name="c", subcore_axis_name="s"),
        scratch_types=[
            pltpu.VMEM((_SPAN,), jnp.float32),
            pltpu.VMEM((_CHUNK,), jnp.int32),
            pltpu.VMEM((_CHUNK,), jnp.float32),
            pltpu.VMEM((_CHUNK,), jnp.float32),
        ],
        compiler_params=pltpu.CompilerParams(needs_layout_passes=False),
    )


def kernel(x, n_instances, W1, b1, W2, b2, W3, b3):
    n_instances = n_instances.astype(jnp.int32)
    total = jnp.sum(n_instances)
    nv = (total + _TR - 1) // _TR
    nv_arr = jnp.reshape(nv, (1,)).astype(jnp.int32)
    W1b = W1.astype(jnp.bfloat16)
    b1r = b1.reshape(1, _H)
    b2r = b2.reshape(1, _D)
    b3p = jnp.pad(b3.reshape(1, 1), ((0, 0), (0, 127)))
    s2d, consts = _tc_stage(nv_arr, x, W1b, b1r, W2, W3, b2r, b3p)
    out16 = _sc_stage()(s2d.reshape(_T), n_instances, consts.reshape(128))
    return out16[:, :1]


# speedup vs baseline: 12.1632x; 1.6573x over previous
import jax
import jax.numpy as jnp
from jax import lax
from jax.experimental import pallas as pl
from jax.experimental.pallas import tpu as pltpu

_T = 32768
_D = 512
_H = 1024
_B = 16
_O = 1
_TR = 1024
_NT = _T // _TR


def _num_valid(n_ref):
    total = n_ref[0]
    for j in range(1, _B):
        total = total + n_ref[j]
    return (total + _TR - 1) // _TR


def _clamped(i, n_ref):
    return jnp.maximum(jnp.minimum(i, _num_valid(n_ref) - 1), 0)


def _tc_body(n_ref, x_ref, w1_ref, b1_ref, w2_ref, w3_ref, b2_ref, b3_ref,
             out_ref, w1b_ref, acc_ref):
    i = pl.program_id(0)

    @pl.when(i == 0)
    def _prologue():
        w1b_ref[...] = w1_ref[...].astype(jnp.bfloat16)
        acc_ref[...] = jnp.zeros((_B, _H), jnp.float32)

    @pl.when(i < _num_valid(n_ref))
    def _tile():
        xb = x_ref[...].astype(jnp.bfloat16)
        g = jnp.dot(xb, w1b_ref[...], preferred_element_type=jnp.float32)
        h = jnp.maximum(g + b1_ref[...], 0.0)
        hb = h.astype(jnp.bfloat16)
        biota = lax.broadcasted_iota(jnp.int32, (_B, 1), 0)
        lo = jnp.zeros((_B, 1), jnp.int32)
        hi = jnp.zeros((_B, 1), jnp.int32)
        pb = None
        for j in range(_B):
            pb = n_ref[j] if pb is None else pb + n_ref[j]
            lo = jnp.where(biota == j + 1, pb, lo)
            hi = jnp.where(biota == j, pb, hi)
        gr = lax.broadcasted_iota(jnp.int32, (_B, _TR), 1) + i * _TR
        mask = ((gr >= lo) & (gr < hi)).astype(jnp.bfloat16)
        acc_ref[...] += jnp.dot(mask, hb,
                                preferred_element_type=jnp.float32)

    @pl.when(i == _NT - 1)
    def _epilogue():
        acc = acc_ref[...]
        hi16 = acc.astype(jnp.bfloat16)
        lo16 = (acc - hi16.astype(jnp.float32)).astype(jnp.bfloat16)
        w2b = w2_ref[...].astype(jnp.bfloat16)
        prod = jnp.dot(jnp.concatenate([hi16, lo16], axis=0), w2b,
                       preferred_element_type=jnp.float32)
        nvals = jnp.zeros((_B, 1), jnp.float32)
        biota = lax.broadcasted_iota(jnp.int32, (_B, 1), 0)
        for j in range(_B):
            nvals = jnp.where(biota == j, n_ref[j].astype(jnp.float32), nvals)
        agg = prod[:_B] + prod[_B:] + nvals * b2_ref[...]
        aggb = agg.astype(jnp.bfloat16)
        w3b = w3_ref[...].astype(jnp.bfloat16)
        out_ref[...] = jnp.dot(aggb, w3b,
                               preferred_element_type=jnp.float32) + b3_ref[...]


def kernel(x, n_instances, W1, b1, W2, b2, W3, b3):
    grid_spec = pltpu.PrefetchScalarGridSpec(
        num_scalar_prefetch=1,
        grid=(_NT,),
        in_specs=[
            pl.BlockSpec((_TR, _D), lambda i, n: (_clamped(i, n), 0)),
            pl.BlockSpec((_D, _H), lambda i, n: (0, 0)),
            pl.BlockSpec((1, _H), lambda i, n: (0, 0)),
            pl.BlockSpec((_H, _D), lambda i, n: (0, 0)),
            pl.BlockSpec((_D, _O), lambda i, n: (0, 0)),
            pl.BlockSpec((1, _D), lambda i, n: (0, 0)),
            pl.BlockSpec((1, _O), lambda i, n: (0, 0)),
        ],
        out_specs=pl.BlockSpec((_B, _O), lambda i, n: (0, 0)),
        scratch_shapes=[
            pltpu.VMEM((_D, _H), jnp.bfloat16),
            pltpu.VMEM((_B, _H), jnp.float32),
        ],
    )
    return pl.pallas_call(
        _tc_body,
        grid_spec=grid_spec,
        out_shape=jax.ShapeDtypeStruct((_B, _O), jnp.float32),
    )(n_instances, x, W1, b1.reshape(1, _H), W2, W3, b2.reshape(1, _D),
      b3.reshape(1, _O))
